# Initial kernel scaffold; baseline (speedup 1.0000x reference)
import jax
import jax.numpy as jnp
from jax.experimental import pallas as pl


def kernel(embeddings, instance_ids):
    raise NotImplementedError("write your pallas kernel here")


# speedup vs baseline: 52.3485x; 52.3485x over previous
import jax
import jax.numpy as jnp
from jax import lax
from jax.experimental import pallas as pl

_DELTA_V = 0.5
_DELTA_D = 1.5
_ALPHA = 1.0
_BETA = 1.0
_GAMMA = 0.001
_K = 64
_N = 32768
_D = 16


def _body(emb_ref, ids_ref, out_ref):
    et = emb_ref[0]
    ids = ids_ref[0, 0]

    seg = lax.broadcasted_iota(jnp.int32, (_N, _K), 1)
    oneh = (ids[:, None] == seg).astype(jnp.float32)

    counts = jnp.sum(oneh, axis=0)
    sums_t = lax.dot_general(et, oneh, (((1,), (0,)), ((), ())),
                             preferred_element_type=jnp.float32)
    denom = jnp.maximum(counts, 1.0)
    invc = 1.0 / denom
    means_t = sums_t * invc[None, :]

    proj_t = lax.dot_general(means_t, oneh, (((1,), (1,)), ((), ())),
                             preferred_element_type=jnp.float32)
    diffs = et - proj_t
    d2 = jnp.sum(diffs * diffs, axis=0, keepdims=True) + 1e-12
    dist = jnp.sqrt(d2)
    hinge = jnp.maximum(dist - _DELTA_V, 0.0) ** 2
    w = lax.dot_general(invc[None, :], oneh, (((1,), (1,)), ((), ())),
                        preferred_element_type=jnp.float32)
    var_loss = jnp.sum(hinge * w) / _K

    md = means_t[:, :, None] - means_t[:, None, :]
    sq = jnp.sum(md * md, axis=0)
    ii = lax.broadcasted_iota(jnp.int32, (_K, _K), 0)
    jj = lax.broadcasted_iota(jnp.int32, (_K, _K), 1)
    iu = jj > ii
    pd = jnp.sqrt(jnp.where(iu, sq, 1.0))
    h = jnp.maximum(2.0 * _DELTA_D - pd, 0.0) ** 2
    num_pairs = _K * (_K - 1) / 2.0
    dist_loss = jnp.sum(jnp.where(iu, h, 0.0)) / num_pairs

    norms = jnp.sqrt(jnp.sum(means_t * means_t, axis=0, keepdims=True) + 1e-12)
    reg_loss = jnp.sum(norms) / _K

    lane = lax.broadcasted_iota(jnp.int32, (1, 1, 128), 2)
    row = jnp.where(lane == 0, var_loss,
                    jnp.where(lane == 1, dist_loss,
                              jnp.where(lane == 2, reg_loss, 0.0)))
    out_ref[...] = row


def kernel(embeddings, instance_ids):
    b = embeddings.shape[0]
    et = embeddings.transpose(0, 2, 1)
    ids3 = instance_ids.reshape(b, 1, _N)
    out = pl.pallas_call(
        _body,
        grid=(b,),
        in_specs=[
            pl.BlockSpec((1, _D, _N), lambda i: (i, 0, 0)),
            pl.BlockSpec((1, 1, _N), lambda i: (i, 0, 0)),
        ],
        out_specs=pl.BlockSpec((1, 1, 128), lambda i: (i, 0, 0)),
        out_shape=jax.ShapeDtypeStruct((b, 1, 128), jnp.float32),
    )(et, ids3)
    vb, db, rb = out[:, 0, 0], out[:, 0, 1], out[:, 0, 2]
    var_loss = jnp.mean(vb)
    dist_loss = jnp.mean(db)
    reg_loss = jnp.mean(rb)
    total = _ALPHA * var_loss + _BETA * dist_loss + _GAMMA * reg_loss
    return (total, var_loss, dist_loss, reg_loss)
